# baseline (speedup 1.0000x reference)
import jax
import jax.numpy as jnp
from jax import lax
from jax.experimental import pallas as pl
from jax.experimental.pallas import tpu as pltpu
from jax.experimental.pallas import tpu_sc as plsc

_BATCH = 16384
_D = 32
_NW = 32
_BPW = _BATCH // _NW
_HALF = _BPW // 2
_NBLK = _HALF // 16


def _sc_body(users_hbm, product_hbm, utab_hbm, ptab_hbm, wb_hbm, bias_hbm,
             out_hbm, uidx_v, pidx_v, uidx4_v, pidx4_v, urows_v, prows_v,
             wb_v, bias_v, out_v, sem_u, sem_p):
    c = lax.axis_index("c")
    s = lax.axis_index("s")
    wid = s * 2 + c
    base = wid * _BPW

    pltpu.sync_copy(users_hbm.at[pl.ds(base, _BPW)], uidx_v)
    pltpu.sync_copy(product_hbm.at[pl.ds(base, _BPW)], pidx_v)
    pltpu.sync_copy(wb_hbm, wb_v)
    pltpu.sync_copy(bias_hbm, bias_v)
    bias = bias_v[...]

    for h in range(2):
        hb = h * _HALF

        def prep(i, carry):
            uv = uidx_v[pl.ds(hb + i * 16, 16)]
            pv = pidx_v[pl.ds(hb + i * 16, 16)]
            uidx4_v[pl.ds(i * 16, 16)] = lax.shift_right_logical(uv, 2)
            pidx4_v[pl.ds(i * 16, 16)] = lax.shift_right_logical(pv, 2)
            return carry

        lax.fori_loop(0, _NBLK, prep, 0)

        cp_u = pltpu.async_copy(utab_hbm.at[uidx4_v], urows_v, sem_u)
        cp_p = pltpu.async_copy(ptab_hbm.at[pidx4_v], prows_v, sem_p)
        cp_u.wait()
        cp_p.wait()

        def blk(j, carry):
            row_ids = j * 16 + lax.iota(jnp.int32, 16)
            uv = uidx_v[pl.ds(hb + j * 16, 16)]
            pv = pidx_v[pl.ds(hb + j * 16, 16)]
            ucol = lax.shift_left(jnp.bitwise_and(uv, 3), 5)
            pcol = lax.shift_left(jnp.bitwise_and(pv, 3), 5)
            acc = bias
            for d in range(_D):
                g = plsc.load_gather(urows_v, [row_ids, ucol + d])
                acc = acc + g * wb_v[d, :]
            for d in range(_D):
                g = plsc.load_gather(prows_v, [row_ids, pcol + d])
                acc = acc + g * wb_v[_D + d, :]
            out_v[pl.ds(hb + j * 16, 16)] = acc
            return carry

        lax.fori_loop(0, _NBLK, blk, 0)

    pltpu.sync_copy(out_v, out_hbm.at[pl.ds(base, _BPW)])


@jax.jit
def _run(users, product, user_table, product_table, W, b):
    wb = jnp.broadcast_to(W.reshape(2 * _D, 1), (2 * _D, 16))
    bb = jnp.broadcast_to(b.reshape(1), (16,))
    utab4 = user_table.reshape(-1, 128)
    ptab4 = product_table.reshape(-1, 128)
    mesh = plsc.VectorSubcoreMesh(core_axis_name="c", subcore_axis_name="s")
    out = pl.kernel(
        _sc_body,
        mesh=mesh,
        out_type=jax.ShapeDtypeStruct((_BATCH,), jnp.float32),
        scratch_types=[
            pltpu.VMEM((_BPW,), jnp.int32),
            pltpu.VMEM((_BPW,), jnp.int32),
            pltpu.VMEM((_HALF,), jnp.int32),
            pltpu.VMEM((_HALF,), jnp.int32),
            pltpu.VMEM((_HALF, 128), jnp.float32),
            pltpu.VMEM((_HALF, 128), jnp.float32),
            pltpu.VMEM((2 * _D, 16), jnp.float32),
            pltpu.VMEM((16,), jnp.float32),
            pltpu.VMEM((_BPW,), jnp.float32),
            pltpu.SemaphoreType.DMA,
            pltpu.SemaphoreType.DMA,
        ],
        compiler_params=pltpu.CompilerParams(
            needs_layout_passes=False, use_tc_tiling_on_sc=False),
    )(users, product, utab4, ptab4, wb, bb)
    return out.reshape(_BATCH, 1)


def kernel(users, product, user_table, product_table, W, b):
    return _run(users, product, user_table, product_table, W, b)


# speedup vs baseline: 4.2743x; 4.2743x over previous
import functools

import jax
import jax.numpy as jnp
from jax import lax
from jax.experimental import pallas as pl
from jax.experimental.pallas import tpu as pltpu
from jax.experimental.pallas import tpu_sc as plsc

_BATCH = 16384
_D = 32
_NW = 32
_BPW = _BATCH // _NW
_NBLK = _BPW // 16
_CHUNK = 8192


def _sweep_body(w_ref, tabT_ref, s_ref):
    s_ref[...] = jnp.sum(tabT_ref[...] * w_ref[...], axis=0)


def _sweep(tabT, w2, n):
    grid = (n + _CHUNK - 1) // _CHUNK
    return pl.pallas_call(
        _sweep_body,
        grid=(grid,),
        in_specs=[
            pl.BlockSpec((_D, 1), lambda i: (0, 0)),
            pl.BlockSpec((_D, _CHUNK), lambda i: (0, i)),
        ],
        out_specs=pl.BlockSpec((_CHUNK,), lambda i: (i,)),
        out_shape=jax.ShapeDtypeStruct((n,), jnp.float32),
    )(w2, tabT)


def _sc_body(users_hbm, product_hbm, su_hbm, sp_hbm, bias_hbm, out_hbm,
             uidx_v, pidx_v, urow_v, prow_v, ug_v, pg_v, bias_v, out_v,
             sem_u, sem_p):
    c = lax.axis_index("c")
    s = lax.axis_index("s")
    wid = s * 2 + c
    base = wid * _BPW

    pltpu.sync_copy(users_hbm.at[pl.ds(base, _BPW)], uidx_v)
    pltpu.sync_copy(product_hbm.at[pl.ds(base, _BPW)], pidx_v)
    pltpu.sync_copy(bias_hbm, bias_v)
    bias = bias_v[...]

    def prep(i, carry):
        uv = uidx_v[pl.ds(i * 16, 16)]
        pv = pidx_v[pl.ds(i * 16, 16)]
        urow_v[pl.ds(i * 16, 16)] = lax.shift_right_logical(uv, 4)
        prow_v[pl.ds(i * 16, 16)] = lax.shift_right_logical(pv, 4)
        return carry

    lax.fori_loop(0, _NBLK, prep, 0)

    cp_u = pltpu.async_copy(su_hbm.at[urow_v], ug_v, sem_u)
    cp_p = pltpu.async_copy(sp_hbm.at[prow_v], pg_v, sem_p)
    cp_u.wait()
    cp_p.wait()

    def blk(j, carry):
        rel = j * 16 + lax.iota(jnp.int32, 16)
        ucol = jnp.bitwise_and(uidx_v[pl.ds(j * 16, 16)], 15)
        pcol = jnp.bitwise_and(pidx_v[pl.ds(j * 16, 16)], 15)
        gu = plsc.load_gather(ug_v, [rel, ucol])
        gp = plsc.load_gather(pg_v, [rel, pcol])
        out_v[pl.ds(j * 16, 16)] = gu + gp + bias
        return carry

    lax.fori_loop(0, _NBLK, blk, 0)
    pltpu.sync_copy(out_v, out_hbm.at[pl.ds(base, _BPW)])


@jax.jit
def _run(users, product, user_table, product_table, W, b):
    n_users = user_table.shape[0]
    n_products = product_table.shape[0]
    wu2 = W[0, :_D].reshape(_D, 1)
    wp2 = W[0, _D:].reshape(_D, 1)
    su = _sweep(user_table.T, wu2, n_users)
    sp = _sweep(product_table.T, wp2, n_products)
    su2d = su.reshape(-1, 16)
    sp2d = sp.reshape(-1, 16)
    bb = jnp.broadcast_to(b.reshape(1), (16,))

    mesh = plsc.VectorSubcoreMesh(core_axis_name="c", subcore_axis_name="s")
    out = pl.kernel(
        _sc_body,
        mesh=mesh,
        out_type=jax.ShapeDtypeStruct((_BATCH,), jnp.float32),
        scratch_types=[
            pltpu.VMEM((_BPW,), jnp.int32),
            pltpu.VMEM((_BPW,), jnp.int32),
            pltpu.VMEM((_BPW,), jnp.int32),
            pltpu.VMEM((_BPW,), jnp.int32),
            pltpu.VMEM((_BPW, 16), jnp.float32),
            pltpu.VMEM((_BPW, 16), jnp.float32),
            pltpu.VMEM((16,), jnp.float32),
            pltpu.VMEM((_BPW,), jnp.float32),
            pltpu.SemaphoreType.DMA,
            pltpu.SemaphoreType.DMA,
        ],
        compiler_params=pltpu.CompilerParams(
            needs_layout_passes=False, use_tc_tiling_on_sc=False),
    )(users, product, su2d, sp2d, bb)
    return out.reshape(_BATCH, 1)


def kernel(users, product, user_table, product_table, W, b):
    return _run(users, product, user_table, product_table, W, b)


# speedup vs baseline: 6.9131x; 1.6174x over previous
import functools

import jax
import jax.numpy as jnp
from jax import lax
from jax.experimental import pallas as pl
from jax.experimental.pallas import tpu as pltpu
from jax.experimental.pallas import tpu_sc as plsc

_BATCH = 16384
_D = 32
_NW = 32
_BPW = _BATCH // _NW
_NBLK = _BPW // 16
_CHUNK = 32768


def _sweep_body(w_ref, tabT_ref, s_ref):
    s_ref[...] = jnp.sum(tabT_ref[...] * w_ref[...], axis=0)


def _sweep(tabT, w2, n):
    grid = (n + _CHUNK - 1) // _CHUNK
    return pl.pallas_call(
        _sweep_body,
        grid=(grid,),
        in_specs=[
            pl.BlockSpec((_D, 1), lambda i: (0, 0)),
            pl.BlockSpec((_D, _CHUNK), lambda i: (0, i)),
        ],
        out_specs=pl.BlockSpec((_CHUNK,), lambda i: (i,)),
        out_shape=jax.ShapeDtypeStruct((n,), jnp.float32),
    )(w2, tabT)


def _sc_body(users_hbm, product_hbm, su_hbm, sp_hbm, bias_hbm, out_hbm,
             uidx_v, pidx_v, urow_v, prow_v, ug_v, pg_v, bias_v, out_v,
             sem_u, sem_p):
    c = lax.axis_index("c")
    s = lax.axis_index("s")
    wid = s * 2 + c
    base = wid * _BPW

    pltpu.sync_copy(users_hbm.at[pl.ds(base, _BPW)], uidx_v)
    pltpu.sync_copy(product_hbm.at[pl.ds(base, _BPW)], pidx_v)
    pltpu.sync_copy(bias_hbm, bias_v)
    bias = bias_v[...]

    def prep(i, carry):
        uv = uidx_v[pl.ds(i * 16, 16)]
        pv = pidx_v[pl.ds(i * 16, 16)]
        urow_v[pl.ds(i * 16, 16)] = lax.shift_right_logical(uv, 4)
        prow_v[pl.ds(i * 16, 16)] = lax.shift_right_logical(pv, 4)
        return carry

    lax.fori_loop(0, _NBLK, prep, 0)

    cp_u = pltpu.async_copy(su_hbm.at[urow_v], ug_v, sem_u)
    cp_p = pltpu.async_copy(sp_hbm.at[prow_v], pg_v, sem_p)
    cp_u.wait()
    cp_p.wait()

    def blk(j, carry):
        rel = j * 16 + lax.iota(jnp.int32, 16)
        ucol = jnp.bitwise_and(uidx_v[pl.ds(j * 16, 16)], 15)
        pcol = jnp.bitwise_and(pidx_v[pl.ds(j * 16, 16)], 15)
        gu = plsc.load_gather(ug_v, [rel, ucol])
        gp = plsc.load_gather(pg_v, [rel, pcol])
        out_v[pl.ds(j * 16, 16)] = gu + gp + bias
        return carry

    lax.fori_loop(0, _NBLK, blk, 0)
    pltpu.sync_copy(out_v, out_hbm.at[pl.ds(base, _BPW)])


@jax.jit
def _run(users, product, user_table, product_table, W, b):
    n_users = user_table.shape[0]
    n_products = product_table.shape[0]
    wu2 = W[0, :_D].reshape(_D, 1)
    wp2 = W[0, _D:].reshape(_D, 1)
    su = _sweep(user_table.T, wu2, n_users)
    sp = _sweep(product_table.T, wp2, n_products)
    su2d = su.reshape(-1, 16)
    sp2d = sp.reshape(-1, 16)
    bb = jnp.broadcast_to(b.reshape(1), (16,))

    mesh = plsc.VectorSubcoreMesh(core_axis_name="c", subcore_axis_name="s")
    out = pl.kernel(
        _sc_body,
        mesh=mesh,
        out_type=jax.ShapeDtypeStruct((_BATCH,), jnp.float32),
        scratch_types=[
            pltpu.VMEM((_BPW,), jnp.int32),
            pltpu.VMEM((_BPW,), jnp.int32),
            pltpu.VMEM((_BPW,), jnp.int32),
            pltpu.VMEM((_BPW,), jnp.int32),
            pltpu.VMEM((_BPW, 16), jnp.float32),
            pltpu.VMEM((_BPW, 16), jnp.float32),
            pltpu.VMEM((16,), jnp.float32),
            pltpu.VMEM((_BPW,), jnp.float32),
            pltpu.SemaphoreType.DMA,
            pltpu.SemaphoreType.DMA,
        ],
        compiler_params=pltpu.CompilerParams(
            needs_layout_passes=False, use_tc_tiling_on_sc=False),
    )(users, product, su2d, sp2d, bb)
    return out.reshape(_BATCH, 1)


def kernel(users, product, user_table, product_table, W, b):
    return _run(users, product, user_table, product_table, W, b)
